# baseline (speedup 1.0000x reference)
import functools

import jax
import jax.numpy as jnp
from jax.experimental import pallas as pl


def _pad_hw(x, p):
    return jnp.pad(x, ((0, 0), (p, p), (p, p), (0, 0)))


def _parity_planes(x):
    n, h, w, c = x.shape
    x = x.reshape(n, h // 2, 2, w // 2, 2, c)
    return x.transpose(0, 2, 4, 1, 3, 5).reshape(n, 4, h // 2, w // 2, c)


def _row_chunk(ho, wo, k):
    budget = 2 * 1024 * 1024
    tr = max(1, budget // (wo * k * 4))
    tr = min(tr, ho)
    while ho % tr:
        tr -= 1
    return tr


def _conv_plane_body(*refs, taps, ho, wo, cin, cout, act, has_res):
    if has_res:
        x_ref, r_ref, w_ref, b_ref, o_ref = refs
    else:
        x_ref, w_ref, b_ref, o_ref = refs
    k = len(taps) * cin
    tr = _row_chunk(ho, wo, k)
    for r in range(0, ho, tr):
        parts = [x_ref[0, p, r + dr:r + dr + tr, dc:dc + wo, :]
                 for (p, dr, dc) in taps]
        xs = jnp.concatenate(parts, axis=-1).reshape(tr * wo, k)
        y = jnp.dot(xs, w_ref[:], preferred_element_type=jnp.float32)
        y = (y + b_ref[0]).reshape(tr, wo, cout)
        if has_res:
            y = y + r_ref[0, r:r + tr]
        if act:
            y = jnp.maximum(y, 0.0)
        o_ref[0, r:r + tr] = y


def _conv_plane(xp, taps, wmat, b, ho, wo, act, res=None):
    n, pn, hq, wq, cin = xp.shape
    cout = wmat.shape[1]
    b2 = b.reshape(1, cout)
    body = functools.partial(_conv_plane_body, taps=taps, ho=ho, wo=wo,
                             cin=cin, cout=cout, act=act,
                             has_res=res is not None)
    in_specs = [pl.BlockSpec((1, pn, hq, wq, cin), lambda i: (i, 0, 0, 0, 0))]
    args = [xp]
    if res is not None:
        in_specs.append(pl.BlockSpec((1, ho, wo, cout), lambda i: (i, 0, 0, 0)))
        args.append(res)
    in_specs.append(pl.BlockSpec(wmat.shape, lambda i: (0, 0)))
    in_specs.append(pl.BlockSpec((1, cout), lambda i: (0, 0)))
    args += [wmat, b2]
    return pl.pallas_call(
        body,
        grid=(n,),
        in_specs=in_specs,
        out_specs=pl.BlockSpec((1, ho, wo, cout), lambda i: (i, 0, 0, 0)),
        out_shape=jax.ShapeDtypeStruct((n, ho, wo, cout), jnp.float32),
    )(*args)


def _mm_body(x_ref, w_ref, b_ref, o_ref):
    o_ref[:] = jnp.maximum(
        jnp.dot(x_ref[:], w_ref[:], preferred_element_type=jnp.float32)
        + b_ref[0], 0.0)


def _first_down_conv(h, w, b):
    n, hh, ww, c = h.shape
    o = w.shape[0]
    ho, wo = hh // 2, ww // 2
    hp = _pad_hw(h, 1)
    cols = [hp[:, kh:kh + 2 * ho:2, kw:kw + 2 * wo:2, :]
            for kh in range(4) for kw in range(4)]
    pat = jnp.concatenate(cols, axis=-1).reshape(n * ho * wo, 16 * c)
    w2 = w.transpose(2, 3, 1, 0).reshape(16 * c, o)
    m = n * ho * wo
    tile = 4096
    while m % tile:
        tile //= 2
    out = pl.pallas_call(
        _mm_body,
        grid=(m // tile,),
        in_specs=[
            pl.BlockSpec((tile, 16 * c), lambda i: (i, 0)),
            pl.BlockSpec((16 * c, o), lambda i: (0, 0)),
            pl.BlockSpec((1, o), lambda i: (0, 0)),
        ],
        out_specs=pl.BlockSpec((tile, o), lambda i: (i, 0)),
        out_shape=jax.ShapeDtypeStruct((m, o), jnp.float32),
    )(pat, w2, b.reshape(1, o))
    return out.reshape(n, ho, wo, o)


_TAPS_DOWN = [((kh % 2) * 2 + (kw % 2), kh // 2, kw // 2)
              for kh in range(4) for kw in range(4)]
_TAPS_3X3 = [(0, kh, kw) for kh in range(3) for kw in range(3)]


def _down_conv(h, w, b):
    n, hh, ww, c = h.shape
    if c < 16:
        return _first_down_conv(h, w, b)
    planes = _parity_planes(_pad_hw(h, 1))
    wmat = w.transpose(2, 3, 1, 0).reshape(16 * c, w.shape[0])
    return _conv_plane(planes, _TAPS_DOWN, wmat, b, hh // 2, ww // 2, act=True)


def _conv3x3(h, w, b, act, res=None):
    n, hh, ww, c = h.shape
    xp = _pad_hw(h, 1)[:, None]
    wmat = w.transpose(2, 3, 1, 0).reshape(9 * c, w.shape[0])
    return _conv_plane(xp, _TAPS_3X3, wmat, b, hh, ww, act, res=res)


def kernel(x, params):
    h = jnp.transpose(x, (0, 2, 3, 1))
    e = []
    for p in params:
        for (w, b) in p['down']:
            h = _down_conv(h, w, b)
        wf, bf = p['final']
        h = _conv3x3(h, wf, bf, act=False)
        nres = len(p['res'])
        for i, (w1, b1, w2, b2) in enumerate(p['res']):
            hh = _conv3x3(h, w1, b1, act=True)
            h = _conv3x3(hh, w2, b2, act=(i == nres - 1), res=h)
        e.append(jnp.transpose(h, (0, 3, 1, 2)))
    return tuple(e)


# speedup vs baseline: 1.0810x; 1.0810x over previous
import functools

import jax
import jax.numpy as jnp
from jax.experimental import pallas as pl
from jax.experimental.pallas import tpu as pltpu


def _pad_hw(x, p):
    return jnp.pad(x, ((0, 0), (p, p), (p, p), (0, 0)))


def _parity_planes(x):
    n, h, w, c = x.shape
    x = x.reshape(n, h // 2, 2, w // 2, 2, c)
    return x.transpose(0, 2, 4, 1, 3, 5).reshape(n, 4, h // 2, w // 2, c)


def _row_chunk(ho, wo, k):
    budget = 2 * 1024 * 1024
    tr = max(1, budget // (wo * k * 4))
    tr = min(tr, ho)
    while ho % tr:
        tr -= 1
    return tr


def _wmat3(w):
    return w.transpose(2, 3, 1, 0).reshape(9 * w.shape[1], w.shape[0])


def _conv_from(src_ref, wm_ref, b_ref, s, cin, cout):
    k = 9 * cin
    tr = _row_chunk(s, s, k)
    for r in range(0, s, tr):
        parts = [src_ref[r + kh:r + kh + tr, kw:kw + s, :]
                 for kh in range(3) for kw in range(3)]
        xs = jnp.concatenate(parts, axis=-1).reshape(tr * s, k)
        y = jnp.dot(xs, wm_ref[:], preferred_element_type=jnp.float32)
        y = (y + b_ref[0]).reshape(tr, s, cout)
        yield r, tr, y


def _zero_border(ref, s, c):
    ref[0:1] = jnp.zeros((1, s + 2, c), jnp.float32)
    ref[s + 1:s + 2] = jnp.zeros((1, s + 2, c), jnp.float32)
    ref[:, 0:1] = jnp.zeros((s + 2, 1, c), jnp.float32)
    ref[:, s + 1:s + 2] = jnp.zeros((s + 2, 1, c), jnp.float32)


def _chain_body(x_ref, wf_ref, bf_ref, w1_ref, b1_ref, w2_ref, b2_ref,
                w3_ref, b3_ref, w4_ref, b4_ref, o_ref, a_ref, h_ref,
                *, s, cin):
    _zero_border(a_ref, s, 128)
    _zero_border(h_ref, s, 64)
    xp = x_ref.at[0, 0]
    for r, tr, y in _conv_from(xp, wf_ref, bf_ref, s, cin, 128):
        a_ref[1 + r:1 + r + tr, 1:1 + s] = y
    for r, tr, y in _conv_from(a_ref, w1_ref, b1_ref, s, 128, 64):
        h_ref[1 + r:1 + r + tr, 1:1 + s] = jnp.maximum(y, 0.0)
    for r, tr, y in _conv_from(h_ref, w2_ref, b2_ref, s, 64, 128):
        a_ref[1 + r:1 + r + tr, 1:1 + s] = a_ref[1 + r:1 + r + tr, 1:1 + s] + y
    for r, tr, y in _conv_from(a_ref, w3_ref, b3_ref, s, 128, 64):
        h_ref[1 + r:1 + r + tr, 1:1 + s] = jnp.maximum(y, 0.0)
    for r, tr, y in _conv_from(h_ref, w4_ref, b4_ref, s, 64, 128):
        o_ref[0, r:r + tr] = jnp.maximum(
            a_ref[1 + r:1 + r + tr, 1:1 + s] + y, 0.0)


def _res_chain(h, wf, bf, res_params):
    n, s, _, cin = h.shape
    (w1, b1, w2, b2), (w3, b3, w4, b4) = res_params
    xp = _pad_hw(h, 1)[:, None]
    mats = [_wmat3(wf), bf.reshape(1, 128), _wmat3(w1), b1.reshape(1, 64),
            _wmat3(w2), b2.reshape(1, 128), _wmat3(w3), b3.reshape(1, 64),
            _wmat3(w4), b4.reshape(1, 128)]
    in_specs = [pl.BlockSpec((1, 1, s + 2, s + 2, cin),
                             lambda i: (i, 0, 0, 0, 0))]
    for m in mats:
        in_specs.append(pl.BlockSpec(m.shape, lambda i: (0, 0)))
    body = functools.partial(_chain_body, s=s, cin=cin)
    return pl.pallas_call(
        body,
        grid=(n,),
        in_specs=in_specs,
        out_specs=pl.BlockSpec((1, s, s, 128), lambda i: (i, 0, 0, 0)),
        out_shape=jax.ShapeDtypeStruct((n, s, s, 128), jnp.float32),
        scratch_shapes=[pltpu.VMEM((s + 2, s + 2, 128), jnp.float32),
                        pltpu.VMEM((s + 2, s + 2, 64), jnp.float32)],
    )(xp, *mats)


def _down_body(x_ref, w_ref, b_ref, o_ref, *, ho, wo, cin):
    cout = w_ref.shape[1]
    k = 16 * cin
    tr = _row_chunk(ho, wo, k)
    for r in range(0, ho, tr):
        parts = [x_ref[0, (kh % 2) * 2 + (kw % 2),
                       r + kh // 2:r + kh // 2 + tr,
                       kw // 2:kw // 2 + wo, :]
                 for kh in range(4) for kw in range(4)]
        xs = jnp.concatenate(parts, axis=-1).reshape(tr * wo, k)
        y = jnp.dot(xs, w_ref[:], preferred_element_type=jnp.float32)
        y = (y + b_ref[0]).reshape(tr, wo, cout)
        o_ref[0, r:r + tr] = jnp.maximum(y, 0.0)


def _down_conv(h, w, b):
    n, hh, ww, c = h.shape
    if c < 16:
        return _first_down_conv(h, w, b)
    ho, wo = hh // 2, ww // 2
    planes = _parity_planes(_pad_hw(h, 1))
    wmat = w.transpose(2, 3, 1, 0).reshape(16 * c, w.shape[0])
    body = functools.partial(_down_body, ho=ho, wo=wo, cin=c)
    return pl.pallas_call(
        body,
        grid=(n,),
        in_specs=[
            pl.BlockSpec((1, 4, ho + 1, wo + 1, c),
                         lambda i: (i, 0, 0, 0, 0)),
            pl.BlockSpec(wmat.shape, lambda i: (0, 0)),
            pl.BlockSpec((1, w.shape[0]), lambda i: (0, 0)),
        ],
        out_specs=pl.BlockSpec((1, ho, wo, w.shape[0]), lambda i: (i, 0, 0, 0)),
        out_shape=jax.ShapeDtypeStruct((n, ho, wo, w.shape[0]), jnp.float32),
    )(planes, wmat, b.reshape(1, w.shape[0]))


def _mm_body(x_ref, w_ref, b_ref, o_ref):
    o_ref[:] = jnp.maximum(
        jnp.dot(x_ref[:], w_ref[:], preferred_element_type=jnp.float32)
        + b_ref[0], 0.0)


def _first_down_conv(h, w, b):
    n, hh, ww, c = h.shape
    o = w.shape[0]
    ho, wo = hh // 2, ww // 2
    hp = _pad_hw(h, 1)
    cols = [hp[:, kh:kh + 2 * ho:2, kw:kw + 2 * wo:2, :]
            for kh in range(4) for kw in range(4)]
    pat = jnp.concatenate(cols, axis=-1).reshape(n * ho * wo, 16 * c)
    w2 = w.transpose(2, 3, 1, 0).reshape(16 * c, o)
    m = n * ho * wo
    tile = 4096
    while m % tile:
        tile //= 2
    out = pl.pallas_call(
        _mm_body,
        grid=(m // tile,),
        in_specs=[
            pl.BlockSpec((tile, 16 * c), lambda i: (i, 0)),
            pl.BlockSpec((16 * c, o), lambda i: (0, 0)),
            pl.BlockSpec((1, o), lambda i: (0, 0)),
        ],
        out_specs=pl.BlockSpec((tile, o), lambda i: (i, 0)),
        out_shape=jax.ShapeDtypeStruct((m, o), jnp.float32),
    )(pat, w2, b.reshape(1, o))
    return out.reshape(n, ho, wo, o)


def kernel(x, params):
    h = jnp.transpose(x, (0, 2, 3, 1))
    e = []
    for p in params:
        for (w, b) in p['down']:
            h = _down_conv(h, w, b)
        wf, bf = p['final']
        h = _res_chain(h, wf, bf, p['res'])
        e.append(jnp.transpose(h, (0, 3, 1, 2)))
    return tuple(e)
